# baseline (speedup 1.0000x reference)
import jax
import jax.numpy as jnp
from jax.experimental import pallas as pl
from jax.experimental.pallas import tpu as pltpu


_SEQ_BLK = 1024


def _add_kernel(x_ref, p_ref, o_ref):
    o_ref[...] = x_ref[...] + p_ref[...]


def kernel(inputs, pos_table):
    batch, seq_len, d_model = inputs.shape
    n_seq = seq_len // _SEQ_BLK
    return pl.pallas_call(
        _add_kernel,
        grid=(n_seq, batch),
        in_specs=[
            pl.BlockSpec((1, _SEQ_BLK, d_model), lambda i, j: (j, i, 0)),
            pl.BlockSpec((_SEQ_BLK, d_model), lambda i, j: (i, 0)),
        ],
        out_specs=pl.BlockSpec((1, _SEQ_BLK, d_model), lambda i, j: (j, i, 0)),
        out_shape=jax.ShapeDtypeStruct(inputs.shape, inputs.dtype),
        compiler_params=pltpu.CompilerParams(
            dimension_semantics=("parallel", "parallel"),
        ),
    )(inputs, pos_table)


# speedup vs baseline: 1.0838x; 1.0838x over previous
import jax
import jax.numpy as jnp
from jax.experimental import pallas as pl
from jax.experimental.pallas import tpu as pltpu


_SEQ_BLK = 2048


def _add_kernel(x_ref, p_ref, o_ref):
    o_ref[...] = x_ref[...] + p_ref[...]


def kernel(inputs, pos_table):
    batch, seq_len, d_model = inputs.shape
    n_seq = seq_len // _SEQ_BLK
    return pl.pallas_call(
        _add_kernel,
        grid=(n_seq, batch),
        in_specs=[
            pl.BlockSpec((1, _SEQ_BLK, d_model), lambda i, j: (j, i, 0)),
            pl.BlockSpec((_SEQ_BLK, d_model), lambda i, j: (i, 0)),
        ],
        out_specs=pl.BlockSpec((1, _SEQ_BLK, d_model), lambda i, j: (j, i, 0)),
        out_shape=jax.ShapeDtypeStruct(inputs.shape, inputs.dtype),
        compiler_params=pltpu.CompilerParams(
            dimension_semantics=("parallel", "parallel"),
        ),
    )(inputs, pos_table)
